# baseline (speedup 1.0000x reference)
import functools

import jax
import jax.numpy as jnp
from jax import lax
from jax.experimental import pallas as pl
from jax.experimental.pallas import tpu as pltpu
from jax.experimental.pallas import tpu_sc as plsc


def kernel(unary, binary, index1, index2):
    N_SLICE = 2
    B_full = binary.shape[0]
    Bs = B_full // N_SLICE
    outs = []
    for s in range(N_SLICE):
        sl = slice(s * Bs, (s + 1) * Bs)
        outs.append(_join_slice(unary, binary[sl], index1[sl], index2[sl]))
    return jnp.concatenate(outs, axis=0)


def _join_slice(unary, binary, index1, index2):
    V, D = unary.shape
    B, F = binary.shape
    out_cols = 2 * D + F

    info = plsc.get_sparse_core_info()
    NC, NS = info.num_cores, info.num_subcores
    NW = NC * NS
    per_w = B // NW
    E = 40
    NBUF = 4
    n_chunks = per_w // E
    steady_iters = (n_chunks - 2 * NBUF) // NBUF + 1
    c0 = steady_iters * NBUF

    mesh = plsc.VectorSubcoreMesh(core_axis_name="c", subcore_axis_name="s")

    @functools.partial(
        pl.kernel,
        mesh=mesh,
        out_type=jax.ShapeDtypeStruct((B, out_cols), jnp.float32),
        scratch_types=[
            [pltpu.VMEM((E,), jnp.int32) for _ in range(NBUF)],
            [pltpu.VMEM((E,), jnp.int32) for _ in range(NBUF)],
            [pltpu.VMEM((E, D), jnp.float32) for _ in range(NBUF)],
            [pltpu.VMEM((E, D), jnp.float32) for _ in range(NBUF)],
            [pltpu.VMEM((E, F), jnp.float32) for _ in range(NBUF)],
            [pltpu.SemaphoreType.DMA for _ in range(NBUF)],
            [pltpu.SemaphoreType.DMA for _ in range(NBUF)],
            [pltpu.SemaphoreType.DMA for _ in range(NBUF)],
        ],
    )
    def join_k(unary_hbm, binary_hbm, idx1_hbm, idx2_hbm, out_hbm,
               idx1b, idx2b, rows1, rows2, binb, isem, gsem, wsem):
        wid = lax.axis_index("s") * NC + lax.axis_index("c")
        base = wid * per_w

        def stage_idx(c, b):
            off = base + c * E
            pltpu.async_copy(idx1_hbm.at[pl.ds(off, E)], idx1b[b], isem[b])
            pltpu.async_copy(idx2_hbm.at[pl.ds(off, E)], idx2b[b], isem[b])

        def wait_idx(b):
            pltpu.make_async_copy(idx1_hbm.at[pl.ds(0, E)], idx1b[b],
                                  isem[b]).wait()
            pltpu.make_async_copy(idx2_hbm.at[pl.ds(0, E)], idx2b[b],
                                  isem[b]).wait()

        def gathers(c, b):
            off = base + c * E
            pltpu.async_copy(unary_hbm.at[idx1b[b]], rows1[b], gsem[b])
            pltpu.async_copy(unary_hbm.at[idx2b[b]], rows2[b], gsem[b])
            pltpu.async_copy(binary_hbm.at[pl.ds(off, E)], binb[b], gsem[b])

        def wait_gathers(b):
            pltpu.make_async_copy(unary_hbm.at[pl.ds(0, E)], rows1[b],
                                  gsem[b]).wait()
            pltpu.make_async_copy(unary_hbm.at[pl.ds(0, E)], rows2[b],
                                  gsem[b]).wait()
            pltpu.make_async_copy(binary_hbm.at[pl.ds(0, E)], binb[b],
                                  gsem[b]).wait()

        def scatters(c, b):
            off = base + c * E
            pltpu.async_copy(rows1[b],
                             out_hbm.at[pl.ds(off, E), pl.ds(0, D)], wsem[b])
            pltpu.async_copy(rows2[b],
                             out_hbm.at[pl.ds(off, E), pl.ds(D, D)], wsem[b])
            pltpu.async_copy(binb[b],
                             out_hbm.at[pl.ds(off, E), pl.ds(2 * D, F)],
                             wsem[b])

        def wait_scatters(b):
            pltpu.make_async_copy(rows1[b],
                                  out_hbm.at[pl.ds(base, E), pl.ds(0, D)],
                                  wsem[b]).wait()
            pltpu.make_async_copy(rows2[b],
                                  out_hbm.at[pl.ds(base, E), pl.ds(D, D)],
                                  wsem[b]).wait()
            pltpu.make_async_copy(binb[b],
                                  out_hbm.at[pl.ds(base, E),
                                             pl.ds(2 * D, F)],
                                  wsem[b]).wait()

        for b in range(NBUF):
            stage_idx(b, b)
        for b in range(NBUF):
            wait_idx(b)
            gathers(b, b)

        def body(i, carry):
            for b in range(NBUF):
                c = i * NBUF + b
                nc = c + NBUF
                wait_gathers(b)
                stage_idx(nc, b)
                scatters(c, b)
                wait_scatters(b)
                wait_idx(b)
                gathers(nc, b)
            return carry

        lax.fori_loop(0, steady_iters, body, 0)

        for c in range(c0, n_chunks):
            b = c % NBUF
            nc = c + NBUF
            wait_gathers(b)
            if nc < n_chunks:
                stage_idx(nc, b)
            scatters(c, b)
            if nc < n_chunks:
                wait_scatters(b)
                wait_idx(b)
                gathers(nc, b)

        for b in range(NBUF):
            wait_scatters(b)

    return join_k(unary, binary, index1, index2)


# speedup vs baseline: 1.3202x; 1.3202x over previous
import functools

import jax
import jax.numpy as jnp
from jax import lax
from jax.experimental import pallas as pl
from jax.experimental.pallas import tpu as pltpu
from jax.experimental.pallas import tpu_sc as plsc


def kernel(unary, binary, index1, index2):
    V, D = unary.shape
    B, F = binary.shape
    out_cols = 2 * D + F

    info = plsc.get_sparse_core_info()
    NC, NS = info.num_cores, info.num_subcores
    NW = NC * NS
    per_w = B // NW
    E = 80
    NBUF = 4
    n_chunks = per_w // E
    steady_iters = (n_chunks - 2 * NBUF) // NBUF + 1
    c0 = steady_iters * NBUF

    mesh = plsc.VectorSubcoreMesh(core_axis_name="c", subcore_axis_name="s")

    @functools.partial(
        pl.kernel,
        mesh=mesh,
        compiler_params=pltpu.CompilerParams(use_tc_tiling_on_sc=True),
        out_type=jax.ShapeDtypeStruct((B, out_cols), jnp.float32),
        scratch_types=[
            [pltpu.VMEM((E,), jnp.int32) for _ in range(NBUF)],
            [pltpu.VMEM((E,), jnp.int32) for _ in range(NBUF)],
            [pltpu.VMEM((E, D), jnp.float32) for _ in range(NBUF)],
            [pltpu.VMEM((E, D), jnp.float32) for _ in range(NBUF)],
            [pltpu.VMEM((E, F), jnp.float32) for _ in range(NBUF)],
            [pltpu.SemaphoreType.DMA for _ in range(NBUF)],
            [pltpu.SemaphoreType.DMA for _ in range(NBUF)],
            [pltpu.SemaphoreType.DMA for _ in range(NBUF)],
        ],
    )
    def join_k(unary_hbm, binary_hbm, idx1_hbm, idx2_hbm, out_hbm,
               idx1b, idx2b, rows1, rows2, binb, isem, gsem, wsem):
        wid = lax.axis_index("s") * NC + lax.axis_index("c")
        base = wid * per_w

        def stage_idx(c, b):
            off = base + c * E
            pltpu.async_copy(idx1_hbm.at[pl.ds(off, E)], idx1b[b], isem[b])
            pltpu.async_copy(idx2_hbm.at[pl.ds(off, E)], idx2b[b], isem[b])

        def wait_idx(b):
            pltpu.make_async_copy(idx1_hbm.at[pl.ds(0, E)], idx1b[b],
                                  isem[b]).wait()
            pltpu.make_async_copy(idx2_hbm.at[pl.ds(0, E)], idx2b[b],
                                  isem[b]).wait()

        def gathers(c, b):
            off = base + c * E
            pltpu.async_copy(unary_hbm.at[idx1b[b]], rows1[b], gsem[b])
            pltpu.async_copy(unary_hbm.at[idx2b[b]], rows2[b], gsem[b])
            pltpu.async_copy(binary_hbm.at[pl.ds(off, E)], binb[b], gsem[b])

        def wait_gathers(b):
            pltpu.make_async_copy(unary_hbm.at[pl.ds(0, E)], rows1[b],
                                  gsem[b]).wait()
            pltpu.make_async_copy(unary_hbm.at[pl.ds(0, E)], rows2[b],
                                  gsem[b]).wait()
            pltpu.make_async_copy(binary_hbm.at[pl.ds(0, E)], binb[b],
                                  gsem[b]).wait()

        def scatters(c, b):
            off = base + c * E
            pltpu.async_copy(rows1[b],
                             out_hbm.at[pl.ds(off, E), pl.ds(0, D)], wsem[b])
            pltpu.async_copy(rows2[b],
                             out_hbm.at[pl.ds(off, E), pl.ds(D, D)], wsem[b])
            pltpu.async_copy(binb[b],
                             out_hbm.at[pl.ds(off, E), pl.ds(2 * D, F)],
                             wsem[b])

        def wait_scatters(b):
            pltpu.make_async_copy(rows1[b],
                                  out_hbm.at[pl.ds(base, E), pl.ds(0, D)],
                                  wsem[b]).wait()
            pltpu.make_async_copy(rows2[b],
                                  out_hbm.at[pl.ds(base, E), pl.ds(D, D)],
                                  wsem[b]).wait()
            pltpu.make_async_copy(binb[b],
                                  out_hbm.at[pl.ds(base, E),
                                             pl.ds(2 * D, F)],
                                  wsem[b]).wait()

        for b in range(NBUF):
            stage_idx(b, b)
        for b in range(NBUF):
            wait_idx(b)
            gathers(b, b)

        def body(i, carry):
            for b in range(NBUF):
                c = i * NBUF + b
                nc = c + NBUF
                wait_gathers(b)
                stage_idx(nc, b)
                scatters(c, b)
                wait_scatters(b)
                wait_idx(b)
                gathers(nc, b)
            return carry

        lax.fori_loop(0, steady_iters, body, 0)

        for c in range(c0, n_chunks):
            b = c % NBUF
            nc = c + NBUF
            wait_gathers(b)
            if nc < n_chunks:
                stage_idx(nc, b)
            scatters(c, b)
            if nc < n_chunks:
                wait_scatters(b)
                wait_idx(b)
                gathers(nc, b)

        for b in range(NBUF):
            wait_scatters(b)

    return join_k(unary, binary, index1, index2)


# speedup vs baseline: 1.3284x; 1.0062x over previous
import functools

import jax
import jax.numpy as jnp
from jax import lax
from jax.experimental import pallas as pl
from jax.experimental.pallas import tpu as pltpu
from jax.experimental.pallas import tpu_sc as plsc


def kernel(unary, binary, index1, index2):
    V, D = unary.shape
    B, F = binary.shape
    out_cols = 2 * D + F

    info = plsc.get_sparse_core_info()
    NC, NS = info.num_cores, info.num_subcores
    NW = NC * NS
    per_w = B // NW
    E = 80
    NBUF = 4
    LAG = 2
    n_chunks = per_w // E

    mesh = plsc.VectorSubcoreMesh(core_axis_name="c", subcore_axis_name="s")

    @functools.partial(
        pl.kernel,
        mesh=mesh,
        out_type=jax.ShapeDtypeStruct((B, out_cols), jnp.float32),
        scratch_types=[
            [pltpu.VMEM((E,), jnp.int32) for _ in range(NBUF)],
            [pltpu.VMEM((E,), jnp.int32) for _ in range(NBUF)],
            [pltpu.VMEM((E, D), jnp.float32) for _ in range(NBUF)],
            [pltpu.VMEM((E, D), jnp.float32) for _ in range(NBUF)],
            [pltpu.VMEM((E, F), jnp.float32) for _ in range(NBUF)],
            [pltpu.SemaphoreType.DMA for _ in range(NBUF)],
            [pltpu.SemaphoreType.DMA for _ in range(NBUF)],
            [pltpu.SemaphoreType.DMA for _ in range(NBUF)],
        ],
    )
    def join_k(unary_hbm, binary_hbm, idx1_hbm, idx2_hbm, out_hbm,
               idx1b, idx2b, rows1, rows2, binb, isem, gsem, wsem):
        wid = lax.axis_index("s") * NC + lax.axis_index("c")
        base = wid * per_w

        def stage_idx(c, b):
            off = base + c * E
            pltpu.async_copy(idx1_hbm.at[pl.ds(off, E)], idx1b[b], isem[b])
            pltpu.async_copy(idx2_hbm.at[pl.ds(off, E)], idx2b[b], isem[b])

        def wait_idx(b):
            pltpu.make_async_copy(idx1_hbm.at[pl.ds(0, E)], idx1b[b],
                                  isem[b]).wait()
            pltpu.make_async_copy(idx2_hbm.at[pl.ds(0, E)], idx2b[b],
                                  isem[b]).wait()

        def gathers(c, b):
            off = base + c * E
            pltpu.async_copy(unary_hbm.at[idx1b[b]], rows1[b], gsem[b])
            pltpu.async_copy(unary_hbm.at[idx2b[b]], rows2[b], gsem[b])
            pltpu.async_copy(binary_hbm.at[pl.ds(off, E)], binb[b], gsem[b])

        def wait_gathers(b):
            pltpu.make_async_copy(unary_hbm.at[pl.ds(0, E)], rows1[b],
                                  gsem[b]).wait()
            pltpu.make_async_copy(unary_hbm.at[pl.ds(0, E)], rows2[b],
                                  gsem[b]).wait()
            pltpu.make_async_copy(binary_hbm.at[pl.ds(0, E)], binb[b],
                                  gsem[b]).wait()

        def scatters(c, b):
            off = base + c * E
            pltpu.async_copy(rows1[b],
                             out_hbm.at[pl.ds(off, E), pl.ds(0, D)], wsem[b])
            pltpu.async_copy(rows2[b],
                             out_hbm.at[pl.ds(off, E), pl.ds(D, D)], wsem[b])
            pltpu.async_copy(binb[b],
                             out_hbm.at[pl.ds(off, E), pl.ds(2 * D, F)],
                             wsem[b])

        def wait_scatters(b):
            pltpu.make_async_copy(rows1[b],
                                  out_hbm.at[pl.ds(base, E), pl.ds(0, D)],
                                  wsem[b]).wait()
            pltpu.make_async_copy(rows2[b],
                                  out_hbm.at[pl.ds(base, E), pl.ds(D, D)],
                                  wsem[b]).wait()
            pltpu.make_async_copy(binb[b],
                                  out_hbm.at[pl.ds(base, E),
                                             pl.ds(2 * D, F)],
                                  wsem[b]).wait()

        def step(c, b, first=False, do_stage=True, do_prefetch=True):
            wait_gathers(b)
            if do_stage:
                stage_idx(c + NBUF, b)
            scatters(c, b)
            if do_prefetch:
                bg = (b + LAG) % NBUF
                if not first:
                    wait_scatters(bg)
                wait_idx(bg)
                gathers(c + LAG, bg)

        for b in range(NBUF):
            stage_idx(b, b)
        for b in range(LAG):
            wait_idx(b)
            gathers(b, b)

        step(0, 0, first=True)
        step(1, 1, first=True)
        step(2, 2)
        step(3, 3)

        steady_iters = (n_chunks - 2 * NBUF) // NBUF

        def body(i, carry):
            for b in range(NBUF):
                c = NBUF + i * NBUF + b
                step(c, b)
            return carry

        lax.fori_loop(0, steady_iters, body, 0)

        for c in range(NBUF + steady_iters * NBUF, n_chunks):
            step(c, c % NBUF,
                 do_stage=(c + NBUF < n_chunks),
                 do_prefetch=(c + LAG < n_chunks))

        for c in range(n_chunks - NBUF, n_chunks):
            wait_scatters(c % NBUF)

    return join_k(unary, binary, index1, index2)
